# baseline (speedup 1.0000x reference)
import functools

import jax
import jax.numpy as jnp
from jax import lax
from jax.experimental import pallas as pl
from jax.experimental.pallas import tpu as pltpu
from jax.experimental.pallas import tpu_sc as plsc

_N = 100000
_S = 16
_D = 32
_DEPTH = 2

_NC = 2
_NS = 16
_NW = _NC * _NS
_NODES_PER_TILE = _N // _NW
_P = 8
_G = _P * _S
_CH = _NODES_PER_TILE // _P
_PT = _NODES_PER_TILE - _CH * _P
_NBUF = 6

_LANES = 16

_PK = 128 // _D
_R = _N // _PK
_BLKR = 1000


def _gather_max_sc(f, nidx, layer):
    mesh = plsc.VectorSubcoreMesh(core_axis_name="c", subcore_axis_name="s")
    layer_base = layer * _N * _S
    tile_idx = _NODES_PER_TILE * _S

    @functools.partial(
        pl.kernel,
        mesh=mesh,
        compiler_params=pltpu.CompilerParams(use_tc_tiling_on_sc=False),
        out_type=jax.ShapeDtypeStruct((_N * _D,), jnp.float32),
        scratch_types=[
            pltpu.VMEM((tile_idx,), jnp.int32),
            pltpu.VMEM((_NBUF, _G, _D), jnp.float32),
            pltpu.VMEM((_NBUF, _P * _D), jnp.float32),
            pltpu.VMEM((_PT * _S, _D), jnp.float32),
            pltpu.VMEM((_PT * _D,), jnp.float32),
        ] + [pltpu.SemaphoreType.DMA] * (2 * _NBUF + 2),
    )
    def k(f_hbm, idx_hbm, out_hbm, idx_v, rows_v, out_v, trows_v, tout_v,
          *sems):
        gsem = sems[:_NBUF]
        osem = sems[_NBUF:2 * _NBUF]
        tgsem, tosem = sems[2 * _NBUF], sems[2 * _NBUF + 1]
        wid = lax.axis_index("s") * _NC + lax.axis_index("c")
        node_base = wid * _NODES_PER_TILE

        pltpu.sync_copy(
            idx_hbm.at[pl.ds(layer_base + node_base * _S, tile_idx)], idx_v)

        def g_start(c, b):
            pltpu.async_copy(
                f_hbm.at[idx_v.at[pl.ds(c * _G, _G)]], rows_v.at[b], gsem[b])

        def g_wait(c, b):
            pltpu.make_async_copy(
                f_hbm.at[idx_v.at[pl.ds(c * _G, _G)]], rows_v.at[b],
                gsem[b]).wait()

        def o_start(c, b):
            pltpu.async_copy(
                out_v.at[b],
                out_hbm.at[pl.ds((node_base + c * _P) * _D, _P * _D)],
                osem[b])

        def o_wait(c, b):
            pltpu.make_async_copy(
                out_v.at[b],
                out_hbm.at[pl.ds((node_base + c * _P) * _D, _P * _D)],
                osem[b]).wait()

        tail_ids = idx_v.at[pl.ds(_CH * _G, _PT * _S)]
        pltpu.async_copy(f_hbm.at[tail_ids], trows_v, tgsem)

        for b in range(_NBUF):
            g_start(b, b)

        def body(i, carry):
            for b in range(_NBUF):
                c = i * _NBUF + b
                g_wait(c, b)

                @pl.when(i > 0)
                def _():
                    o_wait(c - _NBUF, b)

                for p in range(_P):
                    r0 = p * _S
                    a0 = rows_v[b, r0, 0:_LANES]
                    a1 = rows_v[b, r0, _LANES:_D]
                    for s in range(1, _S):
                        a0 = jnp.maximum(a0, rows_v[b, r0 + s, 0:_LANES])
                        a1 = jnp.maximum(a1, rows_v[b, r0 + s, _LANES:_D])
                    out_v[b, p * _D:p * _D + _LANES] = a0
                    out_v[b, p * _D + _LANES:(p + 1) * _D] = a1

                o_start(c, b)

                @pl.when(i < _CH // _NBUF - 1)
                def _():
                    g_start(c + _NBUF, b)
            return carry

        lax.fori_loop(0, _CH // _NBUF, body, 0)

        pltpu.make_async_copy(f_hbm.at[tail_ids], trows_v, tgsem).wait()
        for p in range(_PT):
            r0 = p * _S
            a0 = trows_v[r0, 0:_LANES]
            a1 = trows_v[r0, _LANES:_D]
            for s in range(1, _S):
                a0 = jnp.maximum(a0, trows_v[r0 + s, 0:_LANES])
                a1 = jnp.maximum(a1, trows_v[r0 + s, _LANES:_D])
            tout_v[p * _D:p * _D + _LANES] = a0
            tout_v[p * _D + _LANES:(p + 1) * _D] = a1
        tail_dst = out_hbm.at[pl.ds((node_base + _CH * _P) * _D, _PT * _D)]
        pltpu.async_copy(tout_v, tail_dst, tosem)

        for b in range(_NBUF):
            o_wait(_CH - _NBUF + b, b)
        pltpu.make_async_copy(tout_v, tail_dst, tosem).wait()

    return k(f, nidx)


def _tc_f(embp, wa128, b128):
    def body(e_ref, w_ref, b_ref, f_ref):
        f_ref[...] = jnp.maximum(
            jnp.dot(e_ref[...], w_ref[...],
                    preferred_element_type=jnp.float32) + b_ref[...],
            0.0)

    return pl.pallas_call(
        body,
        grid=(_R // _BLKR,),
        in_specs=[
            pl.BlockSpec((_BLKR, 128), lambda i: (i, 0)),
            pl.BlockSpec((128, 128), lambda i: (0, 0)),
            pl.BlockSpec((1, 128), lambda i: (0, 0)),
        ],
        out_specs=pl.BlockSpec((_BLKR, 128), lambda i: (i, 0)),
        out_shape=jax.ShapeDtypeStruct((_R, 128), jnp.float32),
    )(embp, wa128, b128)


def _tc_update(embp, pooledp, w1128, w2128, wa128, b128, gmat, gmat_t):
    def body(e_ref, p_ref, w1_ref, w2_ref, wa_ref, b_ref, g_ref, gt_ref,
             eo_ref, fo_ref):
        u = jnp.dot(e_ref[...], w1_ref[...],
                    preferred_element_type=jnp.float32)
        u = u + jnp.dot(p_ref[...], w2_ref[...],
                        preferred_element_type=jnp.float32)
        u = jnp.maximum(u, 0.0)
        s = jnp.dot(u * u, g_ref[...], preferred_element_type=jnp.float32)
        d = 1.0 / jnp.maximum(jnp.sqrt(s), 1e-12)
        e = u * jnp.dot(d, gt_ref[...], preferred_element_type=jnp.float32)
        eo_ref[...] = e
        fo_ref[...] = jnp.maximum(
            jnp.dot(e, wa_ref[...],
                    preferred_element_type=jnp.float32) + b_ref[...],
            0.0)

    return pl.pallas_call(
        body,
        grid=(_R // _BLKR,),
        in_specs=[
            pl.BlockSpec((_BLKR, 128), lambda i: (i, 0)),
            pl.BlockSpec((_BLKR, 128), lambda i: (i, 0)),
            pl.BlockSpec((128, 128), lambda i: (0, 0)),
            pl.BlockSpec((128, 128), lambda i: (0, 0)),
            pl.BlockSpec((128, 128), lambda i: (0, 0)),
            pl.BlockSpec((1, 128), lambda i: (0, 0)),
            pl.BlockSpec((128, _PK), lambda i: (0, 0)),
            pl.BlockSpec((_PK, 128), lambda i: (0, 0)),
        ],
        out_specs=[
            pl.BlockSpec((_BLKR, 128), lambda i: (i, 0)),
            pl.BlockSpec((_BLKR, 128), lambda i: (i, 0)),
        ],
        out_shape=[
            jax.ShapeDtypeStruct((_R, 128), jnp.float32),
            jax.ShapeDtypeStruct((_R, 128), jnp.float32),
        ],
    )(embp, pooledp, w1128, w2128, wa128, b128, gmat, gmat_t)


def kernel(neigh_idx, node_features, W_agg, b_agg, W_upd):
    nidx = neigh_idx.astype(jnp.int32).reshape(_DEPTH * _N * _S)

    eye = jnp.eye(_PK, dtype=jnp.float32)
    wa128 = jnp.kron(eye, W_agg.T)
    w1128 = jnp.kron(eye, W_upd[:, :_D].T)
    w2128 = jnp.kron(eye, W_upd[:, _D:].T)
    b128 = jnp.tile(b_agg, _PK).reshape(1, 128)
    gmat = jnp.repeat(eye, _D, axis=0)
    gmat_t = gmat.T

    embp = node_features.reshape(_R, 128)
    f = _tc_f(embp, wa128, b128)
    for k in range(_DEPTH):
        pooled1d = _gather_max_sc(f.reshape(_N, _D), nidx, k)
        pooledp = pooled1d.reshape(_R, 128)
        embp, f = _tc_update(embp, pooledp, w1128, w2128, wa128, b128,
                             gmat, gmat_t)
    return embp.reshape(_N, _D)


# speedup vs baseline: 1.3412x; 1.3412x over previous
import functools

import jax
import jax.numpy as jnp
from jax import lax
from jax.experimental import pallas as pl
from jax.experimental.pallas import tpu as pltpu
from jax.experimental.pallas import tpu_sc as plsc

_N = 100000
_S = 16
_D = 32
_DEPTH = 2

_NC = 2
_NS = 16
_NW = _NC * _NS
_NODES_PER_TILE = _N // _NW
_P = 5
_G = _P * _S
_CH = _NODES_PER_TILE // _P
_NBUF = 5

_LANES = 16

_PK = 128 // _D
_R = _N // _PK
_BLKR = 1000


def _gather_max_sc(f, nidx, layer):
    mesh = plsc.VectorSubcoreMesh(core_axis_name="c", subcore_axis_name="s")
    layer_base = layer * _N * _S
    tile_idx = _NODES_PER_TILE * _S

    @functools.partial(
        pl.kernel,
        mesh=mesh,
        compiler_params=pltpu.CompilerParams(use_tc_tiling_on_sc=False),
        out_type=jax.ShapeDtypeStruct((_N * _D,), jnp.float32),
        scratch_types=[
            pltpu.VMEM((tile_idx,), jnp.int32),
            pltpu.VMEM((_NBUF, _G, _D), jnp.float32),
            pltpu.VMEM((_NBUF, _P * _D), jnp.float32),
        ] + [pltpu.SemaphoreType.DMA] * (2 * _NBUF),
    )
    def k(f_hbm, idx_hbm, out_hbm, idx_v, rows_v, out_v, *sems):
        gsem = sems[:_NBUF]
        osem = sems[_NBUF:2 * _NBUF]
        wid = lax.axis_index("s") * _NC + lax.axis_index("c")
        node_base = wid * _NODES_PER_TILE

        pltpu.sync_copy(
            idx_hbm.at[pl.ds(layer_base + node_base * _S, tile_idx)], idx_v)

        def g_start(c, b):
            pltpu.async_copy(
                f_hbm.at[idx_v.at[pl.ds(c * _G, _G)]], rows_v.at[b], gsem[b])

        def g_wait(c, b):
            pltpu.make_async_copy(
                f_hbm.at[idx_v.at[pl.ds(c * _G, _G)]], rows_v.at[b],
                gsem[b]).wait()

        def o_start(c, b):
            pltpu.async_copy(
                out_v.at[b],
                out_hbm.at[pl.ds((node_base + c * _P) * _D, _P * _D)],
                osem[b])

        def o_wait(c, b):
            pltpu.make_async_copy(
                out_v.at[b],
                out_hbm.at[pl.ds((node_base + c * _P) * _D, _P * _D)],
                osem[b]).wait()

        for b in range(_NBUF):
            g_start(b, b)

        def body(i, carry):
            for b in range(_NBUF):
                c = i * _NBUF + b
                g_wait(c, b)

                @pl.when(i > 0)
                def _():
                    o_wait(c - _NBUF, b)

                for p in range(_P):
                    r0 = p * _S
                    a0 = rows_v[b, r0, 0:_LANES]
                    a1 = rows_v[b, r0, _LANES:_D]
                    for s in range(1, _S):
                        a0 = jnp.maximum(a0, rows_v[b, r0 + s, 0:_LANES])
                        a1 = jnp.maximum(a1, rows_v[b, r0 + s, _LANES:_D])
                    out_v[b, p * _D:p * _D + _LANES] = a0
                    out_v[b, p * _D + _LANES:(p + 1) * _D] = a1

                o_start(c, b)

                @pl.when(i < _CH // _NBUF - 1)
                def _():
                    g_start(c + _NBUF, b)
            return carry

        lax.fori_loop(0, _CH // _NBUF, body, 0)

        for b in range(_NBUF):
            o_wait(_CH - _NBUF + b, b)

    return k(f, nidx)


def _tc_f(embp, wa128, b128):
    def body(e_ref, w_ref, b_ref, f_ref):
        f_ref[...] = jnp.maximum(
            jnp.dot(e_ref[...], w_ref[...],
                    preferred_element_type=jnp.float32) + b_ref[...],
            0.0)

    return pl.pallas_call(
        body,
        grid=(_R // _BLKR,),
        in_specs=[
            pl.BlockSpec((_BLKR, 128), lambda i: (i, 0)),
            pl.BlockSpec((128, 128), lambda i: (0, 0)),
            pl.BlockSpec((1, 128), lambda i: (0, 0)),
        ],
        out_specs=pl.BlockSpec((_BLKR, 128), lambda i: (i, 0)),
        out_shape=jax.ShapeDtypeStruct((_R, 128), jnp.float32),
    )(embp, wa128, b128)


def _tc_update(embp, pooledp, w1128, w2128, wa128, b128, gmat, gmat_t):
    def body(e_ref, p_ref, w1_ref, w2_ref, wa_ref, b_ref, g_ref, gt_ref,
             eo_ref, fo_ref):
        u = jnp.dot(e_ref[...], w1_ref[...],
                    preferred_element_type=jnp.float32)
        u = u + jnp.dot(p_ref[...], w2_ref[...],
                        preferred_element_type=jnp.float32)
        u = jnp.maximum(u, 0.0)
        s = jnp.dot(u * u, g_ref[...], preferred_element_type=jnp.float32)
        d = 1.0 / jnp.maximum(jnp.sqrt(s), 1e-12)
        e = u * jnp.dot(d, gt_ref[...], preferred_element_type=jnp.float32)
        eo_ref[...] = e
        fo_ref[...] = jnp.maximum(
            jnp.dot(e, wa_ref[...],
                    preferred_element_type=jnp.float32) + b_ref[...],
            0.0)

    return pl.pallas_call(
        body,
        grid=(_R // _BLKR,),
        in_specs=[
            pl.BlockSpec((_BLKR, 128), lambda i: (i, 0)),
            pl.BlockSpec((_BLKR, 128), lambda i: (i, 0)),
            pl.BlockSpec((128, 128), lambda i: (0, 0)),
            pl.BlockSpec((128, 128), lambda i: (0, 0)),
            pl.BlockSpec((128, 128), lambda i: (0, 0)),
            pl.BlockSpec((1, 128), lambda i: (0, 0)),
            pl.BlockSpec((128, _PK), lambda i: (0, 0)),
            pl.BlockSpec((_PK, 128), lambda i: (0, 0)),
        ],
        out_specs=[
            pl.BlockSpec((_BLKR, 128), lambda i: (i, 0)),
            pl.BlockSpec((_BLKR, 128), lambda i: (i, 0)),
        ],
        out_shape=[
            jax.ShapeDtypeStruct((_R, 128), jnp.float32),
            jax.ShapeDtypeStruct((_R, 128), jnp.float32),
        ],
    )(embp, pooledp, w1128, w2128, wa128, b128, gmat, gmat_t)


def kernel(neigh_idx, node_features, W_agg, b_agg, W_upd):
    nidx = neigh_idx.astype(jnp.int32).reshape(_DEPTH * _N * _S)

    eye = jnp.eye(_PK, dtype=jnp.float32)
    wa128 = jnp.kron(eye, W_agg.T)
    w1128 = jnp.kron(eye, W_upd[:, :_D].T)
    w2128 = jnp.kron(eye, W_upd[:, _D:].T)
    b128 = jnp.tile(b_agg, _PK).reshape(1, 128)
    gmat = jnp.repeat(eye, _D, axis=0)
    gmat_t = gmat.T

    embp = node_features.reshape(_R, 128)
    f = _tc_f(embp, wa128, b128)
    for k in range(_DEPTH):
        pooled1d = _gather_max_sc(f.reshape(_N, _D), nidx, k)
        pooledp = pooled1d.reshape(_R, 128)
        embp, f = _tc_update(embp, pooledp, w1128, w2128, wa128, b128,
                             gmat, gmat_t)
    return embp.reshape(_N, _D)
